# Initial kernel scaffold; baseline (speedup 1.0000x reference)
import jax
import jax.numpy as jnp
from jax.experimental import pallas as pl


def kernel(input, edge_index, W, b):
    raise NotImplementedError("write your pallas kernel here")


# speedup vs baseline: 4.6075x; 4.6075x over previous
import functools
import math

import jax
import jax.numpy as jnp
from jax import lax
from jax.experimental import pallas as pl
from jax.experimental.pallas import tpu as pltpu
from jax.experimental.pallas import tpu_sc as plsc

N_NODES = 10000
D = 128

NC = 2
NS = 16
NW = NC * NS

CHUNK = 128
ROWS_PER_SUB = 632
N_PAD = NS * ROWS_PER_SUB


@functools.lru_cache(maxsize=None)
def _sc_scatter(n_chunks):
  mesh = plsc.VectorSubcoreMesh(core_axis_name="c", subcore_axis_name="s")

  @functools.partial(
      pl.kernel,
      mesh=mesh,
      out_type=jax.ShapeDtypeStruct((NC, N_PAD, D), jnp.float32),
      scratch_types=[
          pltpu.VMEM((n_chunks, CHUNK), jnp.int32),
          pltpu.VMEM((n_chunks, CHUNK), jnp.int32),
          pltpu.VMEM((1, CHUNK, D), jnp.float32),
          pltpu.VMEM_SHARED((N_PAD, D), jnp.float32),
          pltpu.SemaphoreType.DMA,
          pltpu.SemaphoreType.DMA,
      ],
  )
  def sc_scatter(x_hbm, src_hbm, dst_hbm, zeros_hbm, out_hbm,
                 src_v, dst_v, rows_v, acc_sh, sem0, sem1):
    c = lax.axis_index("c")
    s = lax.axis_index("s")
    wid = s * NC + c

    pltpu.sync_copy(zeros_hbm.at[pl.ds(s * ROWS_PER_SUB, ROWS_PER_SUB)],
                    acc_sh.at[pl.ds(s * ROWS_PER_SUB, ROWS_PER_SUB)])

    pltpu.sync_copy(src_hbm.at[wid], src_v)
    pltpu.sync_copy(dst_hbm.at[wid], dst_v)

    plsc.subcore_barrier()

    def seq_body(j, carry):
      pltpu.async_copy(x_hbm.at[src_v.at[j]], rows_v.at[0], sem0).wait()
      pltpu.sync_copy(rows_v.at[0], acc_sh.at[dst_v.at[j]], add=True)
      return carry

    lax.fori_loop(0, n_chunks, seq_body, 0)

    plsc.subcore_barrier()

    pltpu.sync_copy(acc_sh.at[pl.ds(s * ROWS_PER_SUB, ROWS_PER_SUB)],
                    out_hbm.at[c].at[pl.ds(s * ROWS_PER_SUB, ROWS_PER_SUB)])

  return sc_scatter


BLK = 1000


def _tc_body(p0_ref, p1_ref, w_ref, b_ref, o_ref):
  acc = p0_ref[...] + p1_ref[...]
  o_ref[...] = (
      jnp.dot(acc, w_ref[...], preferred_element_type=jnp.float32) + b_ref[...]
  )


def _tc_finish(p0, p1, W, b):
  grid = (N_NODES // BLK,)
  return pl.pallas_call(
      _tc_body,
      grid=grid,
      in_specs=[
          pl.BlockSpec((BLK, D), lambda i: (i, 0)),
          pl.BlockSpec((BLK, D), lambda i: (i, 0)),
          pl.BlockSpec((D, D), lambda i: (0, 0)),
          pl.BlockSpec((1, D), lambda i: (0, 0)),
      ],
      out_specs=pl.BlockSpec((BLK, D), lambda i: (i, 0)),
      out_shape=jax.ShapeDtypeStruct((N_NODES, D), jnp.float32),
  )(p0, p1, W, b.reshape(1, D))


def kernel(input, edge_index, W, b):
  dst = edge_index[0].astype(jnp.int32)
  src = edge_index[1].astype(jnp.int32)
  E = src.shape[0]
  n_chunks = math.ceil(E / (NW * CHUNK))
  e_pad = NW * n_chunks * CHUNK
  pad = e_pad - E
  if pad:
    src = jnp.concatenate([src, jnp.zeros((pad,), jnp.int32)])
    dst = jnp.concatenate([dst, jnp.full((pad,), N_NODES, jnp.int32)])
  src3 = src.reshape(NW, n_chunks, CHUNK)
  dst3 = dst.reshape(NW, n_chunks, CHUNK)
  zeros = jnp.zeros((N_PAD, D), jnp.float32)

  partials = _sc_scatter(n_chunks)(input, src3, dst3, zeros)
  p = partials[:, :N_NODES]
  return _tc_finish(p[0], p[1], W, b)
